# baseline (speedup 1.0000x reference)
import functools
import jax
import jax.numpy as jnp
from jax import lax
from jax.experimental import pallas as pl
from jax.experimental.pallas import tpu as pltpu
from jax.experimental.pallas import tpu_sc as plsc

D = 64
N_NODES = 5000
N_EDGES = 10000
NPAD = 5120
EPAD = 10240
ET = 512
NT = EPAD // ET

NW = 32
EPC = EPAD // 2
EPW = EPAD // NW
NCH = EPW // 64
RPW = NPAD // 16
D2 = 128

_f32 = jnp.float32
_DNN = (((1,), (0,)), ((), ()))
_DNT = (((1,), (1,)), ((), ()))


def _mm(a, b):
    return lax.dot_general(a, b, _DNN, preferred_element_type=_f32)


def _mmt(a, b):
    return lax.dot_general(a, b, _DNT, preferred_element_type=_f32)


def _vmesh():
    return plsc.VectorSubcoreMesh(core_axis_name="c", subcore_axis_name="s")


def _sc_gather(h, idx_flat):
    @functools.partial(
        pl.kernel,
        out_type=jax.ShapeDtypeStruct((EPAD, D2), _f32),
        mesh=_vmesh(),
        scratch_types=[pltpu.VMEM((64,), jnp.int32) for _ in range(NCH)]
        + [pltpu.VMEM((EPW, D2), _f32), pltpu.SemaphoreType.DMA],
    )
    def gk(h_hbm, i_hbm, o_hbm, i0, i1, i2, i3, i4, rows, sem):
        c = lax.axis_index("c")
        s = lax.axis_index("s")
        base = (c * 16 + s) * EPW
        idxs = (i0, i1, i2, i3, i4)
        for j in range(NCH):
            pltpu.sync_copy(i_hbm.at[pl.ds(base + j * 64, 64)], idxs[j])
        for j in range(NCH):
            pltpu.async_copy(h_hbm.at[idxs[j]],
                             rows.at[pl.ds(j * 64, 64)], sem).wait()
        pltpu.sync_copy(rows, o_hbm.at[pl.ds(base, EPW)])

    return gk(h, idx_flat)


def _sc_scatter(msg, dst_flat, zeros_hbm):
    @functools.partial(
        pl.kernel,
        out_type=jax.ShapeDtypeStruct((2 * NPAD, D2), _f32),
        mesh=_vmesh(),
        scratch_types=[pltpu.VMEM_SHARED((NPAD, D2), _f32),
                       pltpu.VMEM((EPW, D2), _f32)]
        + [pltpu.VMEM((64,), jnp.int32) for _ in range(NCH)],
    )
    def sk(m_hbm, i_hbm, z_hbm, o_hbm, acc_sh, mbuf, i0, i1, i2, i3, i4):
        c = lax.axis_index("c")
        s = lax.axis_index("s")
        base = (c * 16 + s) * EPW
        idxs = (i0, i1, i2, i3, i4)
        pltpu.sync_copy(z_hbm.at[pl.ds(s * RPW, RPW)],
                        acc_sh.at[pl.ds(s * RPW, RPW)])
        plsc.subcore_barrier()
        pltpu.sync_copy(m_hbm.at[pl.ds(base, EPW)], mbuf)
        for j in range(NCH):
            pltpu.sync_copy(i_hbm.at[pl.ds(base + j * 64, 64)], idxs[j])
        for j in range(NCH):
            pltpu.sync_copy(mbuf.at[pl.ds(j * 64, 64)],
                            acc_sh.at[idxs[j]], add=True)
        plsc.subcore_barrier()
        pltpu.sync_copy(acc_sh.at[pl.ds(s * RPW, RPW)],
                        o_hbm.at[pl.ds(c * NPAD + s * RPW, RPW)])

    return sk(msg, dst_flat, zeros_hbm)


def _prep_kernel(xp_ref, eaT_ref, dst3_ref, Wl0_ref, bl0_ref, We1T_ref,
                 be1_ref, h0_ref, hidT3_ref, rdeg_ref):
    h0_ref[:, 0:D] = jax.nn.relu(_mm(xp_ref[:], Wl0_ref[:]) + bl0_ref[:])
    h0_ref[:, D:D2] = jnp.zeros((NPAD, D2 - D), _f32)
    eaT = eaT_ref[:]
    for i in range(NT):
        hidT3_ref[i] = jax.nn.relu(
            _mm(We1T_ref[:], eaT[:, i * ET:(i + 1) * ET]) + be1_ref[:])
    rdeg_ref[:] = jnp.zeros((NPAD, 1), _f32)
    ones_row = jnp.ones((1, ET), _f32)

    def deg_body(i, c):
        dst_row = dst3_ref[i]
        niota = lax.broadcasted_iota(jnp.int32, (NPAD, ET), 0)
        oh = jnp.where(dst_row == niota, 1.0, 0.0)
        rdeg_ref[:] = rdeg_ref[:] + _mmt(oh, ones_row)
        return c

    lax.fori_loop(0, NT, deg_body, 0)
    rdeg_ref[:] = 1.0 / jnp.maximum(rdeg_ref[:], 1.0)


def _msg_kernel(os_ref, hidT3_ref, W2pT_ref, B2T_ref, msg_ref, UT_ref):
    def body(i, c):
        os = os_ref[pl.ds(i * ET, ET), 0:D]
        UT_ref[:] = _mmt(W2pT_ref[:], os)
        msgT = _mmt(B2T_ref[:], os)
        hidT = hidT3_ref[i]
        for k in range(D):
            msgT = msgT + hidT[k:k + 1, :] * UT_ref[k * D:(k + 1) * D, :]
        msg_ref[pl.ds(i * ET, ET), 0:D] = msgT.T
        msg_ref[pl.ds(i * ET, ET), D:D2] = jnp.zeros((ET, D2 - D), _f32)
        return c

    lax.fori_loop(0, NT, body, 0)


def _gru_kernel(agg2_ref, rdeg_ref, h_ref, Wr_ref, bcv_ref, Wig_ref, Whg_ref,
                big_ref, bhg_ref, ho_ref):
    h = h_ref[:, 0:D]
    agg = (agg2_ref[pl.ds(0, NPAD), 0:D] + agg2_ref[pl.ds(NPAD, NPAD), 0:D])
    m = jax.nn.relu(agg * rdeg_ref[:] + _mm(h, Wr_ref[:]) + bcv_ref[:])
    gi = _mm(m, Wig_ref[:]) + big_ref[:]
    gh = _mm(h, Whg_ref[:]) + bhg_ref[:]
    r = jax.nn.sigmoid(gi[:, 0:D] + gh[:, 0:D])
    z = jax.nn.sigmoid(gi[:, D:2 * D] + gh[:, D:2 * D])
    cand = jnp.tanh(gi[:, 2 * D:3 * D] + r * gh[:, 2 * D:3 * D])
    ho_ref[:, 0:D] = (1.0 - z) * cand + z * h
    ho_ref[:, D:D2] = jnp.zeros((NPAD, D2 - D), _f32)


def _tail_kernel(h_ref, Wis_ref, Whs_ref, bis_ref, bhs_ref, Wim_ref, Whm_ref,
                 bim_ref, bhm_ref, Wl1_ref, bl1_ref, Wl3_ref, bl3_ref,
                 v_ref, hx_ref, cx_ref):
    def lstm(x, hs, cs, Wi, Wh, bi, bh):
        g = _mm(x, Wi) + bi + _mm(hs, Wh) + bh
        ii = jax.nn.sigmoid(g[:, 0:D])
        ff = jax.nn.sigmoid(g[:, D:2 * D])
        gg = jnp.tanh(g[:, 2 * D:3 * D])
        oo = jax.nn.sigmoid(g[:, 3 * D:4 * D])
        cs = ff * cs + ii * gg
        hs = oo * jnp.tanh(cs)
        return hs, cs

    out = h_ref[:, 0:D]
    smask = lax.broadcasted_iota(jnp.int32, (NPAD, 1), 0) < N_NODES
    q_star = jnp.zeros((1, 2 * D), _f32)
    hs = jnp.zeros((1, D), _f32)
    cs = jnp.zeros((1, D), _f32)
    for _ in range(6):
        hs, cs = lstm(q_star, hs, cs, Wis_ref[:], Whs_ref[:],
                      bis_ref[:], bhs_ref[:])
        q = hs
        e = jnp.sum(out * q, axis=1, keepdims=True)
        e = jnp.where(smask, e, -1e30)
        mx = jnp.max(e, axis=0, keepdims=True)
        a = jnp.exp(e - mx)
        a = jnp.where(smask, a, 0.0)
        ssum = jnp.sum(a, axis=0, keepdims=True)
        a = a / ssum
        rvec = jnp.sum(out * a, axis=0, keepdims=True)
        q_star = jnp.concatenate([q, rvec], axis=1)

    hx, cx = lstm(q_star, jnp.zeros((1, D), _f32), jnp.zeros((1, D), _f32),
                  Wim_ref[:], Whm_ref[:], bim_ref[:], bhm_ref[:])
    o1 = jax.nn.relu(_mm(hx, Wl1_ref[:]) + bl1_ref[:])
    v_ref[:] = _mm(o1, Wl3_ref[:]) + bl3_ref[:]
    hx_ref[:] = hx
    cx_ref[:] = cx


def kernel(x, edge_index, edge_attr, batch, W_lin0, b_lin0, W_e1, b_e1,
           W_e2, b_e2, W_root, b_conv, W_ih_gru, W_hh_gru, b_ih_gru, b_hh_gru,
           W_ih_s2s, W_hh_s2s, b_ih_s2s, b_hh_s2s, W_ih_mem, W_hh_mem,
           b_ih_mem, b_hh_mem, W_lin1, b_lin1, W_lin3, b_lin3):
    xp = jnp.zeros((NPAD, 8), _f32).at[:N_NODES, :3].set(x)
    src = edge_index[0].astype(jnp.int32)
    dst = edge_index[1].astype(jnp.int32)
    src_p = jnp.zeros((EPAD,), jnp.int32).at[:N_EDGES].set(src)
    dst_p = jnp.full((EPAD,), NPAD - 1, jnp.int32).at[:N_EDGES].set(dst)
    dst3 = dst_p.reshape(NT, 1, ET)
    eaT = jnp.zeros((8, EPAD), _f32).at[:7, :N_EDGES].set(edge_attr.T)
    zeros_hbm = jnp.zeros((NPAD, D2), _f32)

    row = lambda b: b.reshape(1, -1).astype(_f32)
    Wl0 = jnp.zeros((8, D), _f32).at[:3, :].set(W_lin0)
    We1T = jnp.zeros((D, 8), _f32).at[:, :7].set(W_e1.T)
    W2pT = W_e2.reshape(D, D, D).transpose(0, 2, 1).reshape(D * D, D)
    B2T = b_e2.reshape(D, D).T

    h, hidT3, rdeg = pl.pallas_call(
        _prep_kernel,
        out_shape=[
            jax.ShapeDtypeStruct((NPAD, D2), _f32),
            jax.ShapeDtypeStruct((NT, D, ET), _f32),
            jax.ShapeDtypeStruct((NPAD, 1), _f32),
        ],
    )(xp, eaT, dst3, Wl0, row(b_lin0), We1T, b_e1.reshape(-1, 1))

    msg_call = pl.pallas_call(
        _msg_kernel,
        out_shape=jax.ShapeDtypeStruct((EPAD, D2), _f32),
        scratch_shapes=[pltpu.VMEM((D * D, ET), _f32)],
    )
    gru_call = pl.pallas_call(
        _gru_kernel,
        out_shape=jax.ShapeDtypeStruct((NPAD, D2), _f32),
    )

    for _ in range(6):
        out_src = _sc_gather(h, src_p)
        msg = msg_call(out_src, hidT3, W2pT, B2T)
        agg2 = _sc_scatter(msg, dst_p, zeros_hbm)
        h = gru_call(agg2, rdeg, h, W_root.astype(_f32), row(b_conv),
                     W_ih_gru.astype(_f32), W_hh_gru.astype(_f32),
                     row(b_ih_gru), row(b_hh_gru))

    v, hx, cx = pl.pallas_call(
        _tail_kernel,
        out_shape=[
            jax.ShapeDtypeStruct((1, 1), _f32),
            jax.ShapeDtypeStruct((1, D), _f32),
            jax.ShapeDtypeStruct((1, D), _f32),
        ],
    )(h, W_ih_s2s.astype(_f32), W_hh_s2s.astype(_f32), row(b_ih_s2s),
      row(b_hh_s2s), W_ih_mem.astype(_f32), W_hh_mem.astype(_f32),
      row(b_ih_mem), row(b_hh_mem), W_lin1.astype(_f32), row(b_lin1),
      W_lin3.astype(_f32), row(b_lin3))

    return (v, hx.reshape(1, 1, D), cx.reshape(1, 1, D))


# speedup vs baseline: 1.0790x; 1.0790x over previous
import functools
import jax
import jax.numpy as jnp
from jax import lax
from jax.experimental import pallas as pl
from jax.experimental.pallas import tpu as pltpu
from jax.experimental.pallas import tpu_sc as plsc

D = 64
N_NODES = 5000
N_EDGES = 10000
NPAD = 5120
EPAD = 10240
ET = 512
NT = EPAD // ET

NW = 32
EPC = EPAD // 2
EPW = EPAD // NW
CHS = (128, 128, 64)
COF = (0, 128, 256)
RPW = NPAD // 16
D2 = 128

_f32 = jnp.float32
_DNN = (((1,), (0,)), ((), ()))
_DNT = (((1,), (1,)), ((), ()))


def _mm(a, b):
    return lax.dot_general(a, b, _DNN, preferred_element_type=_f32)


def _mmt(a, b):
    return lax.dot_general(a, b, _DNT, preferred_element_type=_f32)


def _vmesh():
    return plsc.VectorSubcoreMesh(core_axis_name="c", subcore_axis_name="s")


def _sc_gather(h, idx_flat):
    @functools.partial(
        pl.kernel,
        out_type=jax.ShapeDtypeStruct((EPAD, D2), _f32),
        mesh=_vmesh(),
        scratch_types=[pltpu.VMEM((n,), jnp.int32) for n in CHS]
        + [pltpu.VMEM((EPW, D2), _f32), pltpu.SemaphoreType.DMA],
    )
    def gk(h_hbm, i_hbm, o_hbm, i0, i1, i2, rows, sem):
        c = lax.axis_index("c")
        s = lax.axis_index("s")
        base = (c * 16 + s) * EPW
        idxs = (i0, i1, i2)
        hs = [pltpu.async_copy(i_hbm.at[pl.ds(base + COF[j], CHS[j])],
                               idxs[j], sem) for j in range(3)]
        for h_ in hs:
            h_.wait()
        hs = [pltpu.async_copy(h_hbm.at[idxs[j]],
                               rows.at[pl.ds(COF[j], CHS[j])], sem)
              for j in range(3)]
        for h_ in hs:
            h_.wait()
        pltpu.sync_copy(rows, o_hbm.at[pl.ds(base, EPW)])

    return gk(h, idx_flat)


def _sc_scatter(msg, dst_flat, zeros_hbm):
    @functools.partial(
        pl.kernel,
        out_type=jax.ShapeDtypeStruct((2 * NPAD, D2), _f32),
        mesh=_vmesh(),
        scratch_types=[pltpu.VMEM_SHARED((NPAD, D2), _f32),
                       pltpu.VMEM((EPW, D2), _f32)]
        + [pltpu.VMEM((n,), jnp.int32) for n in CHS]
        + [pltpu.SemaphoreType.DMA],
    )
    def sk(m_hbm, i_hbm, z_hbm, o_hbm, acc_sh, mbuf, i0, i1, i2, sem):
        c = lax.axis_index("c")
        s = lax.axis_index("s")
        base = (c * 16 + s) * EPW
        idxs = (i0, i1, i2)
        hs = [pltpu.async_copy(z_hbm.at[pl.ds(s * RPW, RPW)],
                               acc_sh.at[pl.ds(s * RPW, RPW)], sem),
              pltpu.async_copy(m_hbm.at[pl.ds(base, EPW)], mbuf, sem)]
        hs += [pltpu.async_copy(i_hbm.at[pl.ds(base + COF[j], CHS[j])],
                                idxs[j], sem) for j in range(3)]
        for h_ in hs:
            h_.wait()
        plsc.subcore_barrier()
        for j in range(3):
            pltpu.sync_copy(mbuf.at[pl.ds(COF[j], CHS[j])],
                            acc_sh.at[idxs[j]], add=True)
        plsc.subcore_barrier()
        pltpu.sync_copy(acc_sh.at[pl.ds(s * RPW, RPW)],
                        o_hbm.at[pl.ds(c * NPAD + s * RPW, RPW)])

    return sk(msg, dst_flat, zeros_hbm)


def _prep_kernel(xp_ref, eaT_ref, dst3_ref, Wl0_ref, bl0_ref, We1T_ref,
                 be1_ref, h0_ref, hidT3_ref, rdeg_ref):
    h0_ref[:, 0:D] = jax.nn.relu(_mm(xp_ref[:], Wl0_ref[:]) + bl0_ref[:])
    h0_ref[:, D:D2] = jnp.zeros((NPAD, D2 - D), _f32)
    eaT = eaT_ref[:]
    for i in range(NT):
        hidT3_ref[i] = jax.nn.relu(
            _mm(We1T_ref[:], eaT[:, i * ET:(i + 1) * ET]) + be1_ref[:])
    rdeg_ref[:] = jnp.zeros((NPAD, 1), _f32)
    ones_row = jnp.ones((1, ET), _f32)

    def deg_body(i, c):
        dst_row = dst3_ref[i]
        niota = lax.broadcasted_iota(jnp.int32, (NPAD, ET), 0)
        oh = jnp.where(dst_row == niota, 1.0, 0.0)
        rdeg_ref[:] = rdeg_ref[:] + _mmt(oh, ones_row)
        return c

    lax.fori_loop(0, NT, deg_body, 0)
    rdeg_ref[:] = 1.0 / jnp.maximum(rdeg_ref[:], 1.0)


def _msg_kernel(os_ref, hidT3_ref, W2pT_ref, B2T_ref, msg_ref, UT_ref):
    def body(i, c):
        os = os_ref[pl.ds(i * ET, ET), 0:D]
        UT_ref[:] = _mmt(W2pT_ref[:], os)
        msgT = _mmt(B2T_ref[:], os)
        hidT = hidT3_ref[i]
        for k in range(D):
            msgT = msgT + hidT[k:k + 1, :] * UT_ref[k * D:(k + 1) * D, :]
        msg_ref[pl.ds(i * ET, ET), 0:D] = msgT.T
        msg_ref[pl.ds(i * ET, ET), D:D2] = jnp.zeros((ET, D2 - D), _f32)
        return c

    lax.fori_loop(0, NT, body, 0)


def _gru_kernel(agg2_ref, rdeg_ref, h_ref, Wr_ref, bcv_ref, Wig_ref, Whg_ref,
                big_ref, bhg_ref, ho_ref):
    h = h_ref[:, 0:D]
    agg = (agg2_ref[pl.ds(0, NPAD), 0:D] + agg2_ref[pl.ds(NPAD, NPAD), 0:D])
    m = jax.nn.relu(agg * rdeg_ref[:] + _mm(h, Wr_ref[:]) + bcv_ref[:])
    gi = _mm(m, Wig_ref[:]) + big_ref[:]
    gh = _mm(h, Whg_ref[:]) + bhg_ref[:]
    r = jax.nn.sigmoid(gi[:, 0:D] + gh[:, 0:D])
    z = jax.nn.sigmoid(gi[:, D:2 * D] + gh[:, D:2 * D])
    cand = jnp.tanh(gi[:, 2 * D:3 * D] + r * gh[:, 2 * D:3 * D])
    ho_ref[:, 0:D] = (1.0 - z) * cand + z * h
    ho_ref[:, D:D2] = jnp.zeros((NPAD, D2 - D), _f32)


def _tail_kernel(h_ref, Wis_ref, Whs_ref, bis_ref, bhs_ref, Wim_ref, Whm_ref,
                 bim_ref, bhm_ref, Wl1_ref, bl1_ref, Wl3_ref, bl3_ref,
                 v_ref, hx_ref, cx_ref):
    def lstm(x, hs, cs, Wi, Wh, bi, bh):
        g = _mm(x, Wi) + bi + _mm(hs, Wh) + bh
        ii = jax.nn.sigmoid(g[:, 0:D])
        ff = jax.nn.sigmoid(g[:, D:2 * D])
        gg = jnp.tanh(g[:, 2 * D:3 * D])
        oo = jax.nn.sigmoid(g[:, 3 * D:4 * D])
        cs = ff * cs + ii * gg
        hs = oo * jnp.tanh(cs)
        return hs, cs

    out = h_ref[:, 0:D]
    smask = lax.broadcasted_iota(jnp.int32, (NPAD, 1), 0) < N_NODES
    q_star = jnp.zeros((1, 2 * D), _f32)
    hs = jnp.zeros((1, D), _f32)
    cs = jnp.zeros((1, D), _f32)
    for _ in range(6):
        hs, cs = lstm(q_star, hs, cs, Wis_ref[:], Whs_ref[:],
                      bis_ref[:], bhs_ref[:])
        q = hs
        e = jnp.sum(out * q, axis=1, keepdims=True)
        e = jnp.where(smask, e, -1e30)
        mx = jnp.max(e, axis=0, keepdims=True)
        a = jnp.exp(e - mx)
        a = jnp.where(smask, a, 0.0)
        ssum = jnp.sum(a, axis=0, keepdims=True)
        a = a / ssum
        rvec = jnp.sum(out * a, axis=0, keepdims=True)
        q_star = jnp.concatenate([q, rvec], axis=1)

    hx, cx = lstm(q_star, jnp.zeros((1, D), _f32), jnp.zeros((1, D), _f32),
                  Wim_ref[:], Whm_ref[:], bim_ref[:], bhm_ref[:])
    o1 = jax.nn.relu(_mm(hx, Wl1_ref[:]) + bl1_ref[:])
    v_ref[:] = _mm(o1, Wl3_ref[:]) + bl3_ref[:]
    hx_ref[:] = hx
    cx_ref[:] = cx


def kernel(x, edge_index, edge_attr, batch, W_lin0, b_lin0, W_e1, b_e1,
           W_e2, b_e2, W_root, b_conv, W_ih_gru, W_hh_gru, b_ih_gru, b_hh_gru,
           W_ih_s2s, W_hh_s2s, b_ih_s2s, b_hh_s2s, W_ih_mem, W_hh_mem,
           b_ih_mem, b_hh_mem, W_lin1, b_lin1, W_lin3, b_lin3):
    xp = jnp.zeros((NPAD, 8), _f32).at[:N_NODES, :3].set(x)
    src = edge_index[0].astype(jnp.int32)
    dst = edge_index[1].astype(jnp.int32)
    src_p = jnp.zeros((EPAD,), jnp.int32).at[:N_EDGES].set(src)
    dst_p = jnp.full((EPAD,), NPAD - 1, jnp.int32).at[:N_EDGES].set(dst)
    dst3 = dst_p.reshape(NT, 1, ET)
    eaT = jnp.zeros((8, EPAD), _f32).at[:7, :N_EDGES].set(edge_attr.T)
    zeros_hbm = jnp.zeros((NPAD, D2), _f32)

    row = lambda b: b.reshape(1, -1).astype(_f32)
    Wl0 = jnp.zeros((8, D), _f32).at[:3, :].set(W_lin0)
    We1T = jnp.zeros((D, 8), _f32).at[:, :7].set(W_e1.T)
    W2pT = W_e2.reshape(D, D, D).transpose(0, 2, 1).reshape(D * D, D)
    B2T = b_e2.reshape(D, D).T

    h, hidT3, rdeg = pl.pallas_call(
        _prep_kernel,
        out_shape=[
            jax.ShapeDtypeStruct((NPAD, D2), _f32),
            jax.ShapeDtypeStruct((NT, D, ET), _f32),
            jax.ShapeDtypeStruct((NPAD, 1), _f32),
        ],
    )(xp, eaT, dst3, Wl0, row(b_lin0), We1T, b_e1.reshape(-1, 1))

    msg_call = pl.pallas_call(
        _msg_kernel,
        out_shape=jax.ShapeDtypeStruct((EPAD, D2), _f32),
        scratch_shapes=[pltpu.VMEM((D * D, ET), _f32)],
    )
    gru_call = pl.pallas_call(
        _gru_kernel,
        out_shape=jax.ShapeDtypeStruct((NPAD, D2), _f32),
    )

    for _ in range(6):
        out_src = _sc_gather(h, src_p)
        msg = msg_call(out_src, hidT3, W2pT, B2T)
        agg2 = _sc_scatter(msg, dst_p, zeros_hbm)
        h = gru_call(agg2, rdeg, h, W_root.astype(_f32), row(b_conv),
                     W_ih_gru.astype(_f32), W_hh_gru.astype(_f32),
                     row(b_ih_gru), row(b_hh_gru))

    v, hx, cx = pl.pallas_call(
        _tail_kernel,
        out_shape=[
            jax.ShapeDtypeStruct((1, 1), _f32),
            jax.ShapeDtypeStruct((1, D), _f32),
            jax.ShapeDtypeStruct((1, D), _f32),
        ],
    )(h, W_ih_s2s.astype(_f32), W_hh_s2s.astype(_f32), row(b_ih_s2s),
      row(b_hh_s2s), W_ih_mem.astype(_f32), W_hh_mem.astype(_f32),
      row(b_ih_mem), row(b_hh_mem), W_lin1.astype(_f32), row(b_lin1),
      W_lin3.astype(_f32), row(b_lin3))

    return (v, hx.reshape(1, 1, D), cx.reshape(1, 1, D))
